# Initial kernel scaffold; baseline (speedup 1.0000x reference)
import jax
import jax.numpy as jnp
from jax.experimental import pallas as pl


def kernel(inputs, W):
    raise NotImplementedError("write your pallas kernel here")


# speedup vs baseline: 1.6549x; 1.6549x over previous
import jax
import jax.numpy as jnp
from jax.experimental import pallas as pl
from jax.experimental.pallas import tpu as pltpu


def _vq_body(x_ref, w_ref, q_ref, idx_ref):
    x = x_ref[0]
    w = w_ref[...]
    K = w.shape[0]
    TT = x.shape[1]

    s2 = jax.lax.dot_general(
        -2.0 * w, x, (((1,), (0,)), ((), ())),
        preferred_element_type=jnp.float32)
    wn = jnp.sum(w * w, axis=1, keepdims=True)
    xn = jnp.sum(x * x, axis=0, keepdims=True)
    dist = (wn + xn) + s2

    m = jnp.min(dist, axis=0, keepdims=True)
    iota = jax.lax.broadcasted_iota(jnp.int32, (K, TT), 0)
    idx = jnp.min(jnp.where(dist == m, iota, K), axis=0, keepdims=True)

    oh = (iota == idx).astype(jnp.float32)
    q = jax.lax.dot_general(
        w, oh, (((0,), (0,)), ((), ())),
        preferred_element_type=jnp.float32)

    q_ref[0] = x + (q - x)
    idx_ref[0] = idx


def kernel(inputs, W):
    B, D, T = inputs.shape
    K = W.shape[0]
    q, idx = pl.pallas_call(
        _vq_body,
        grid=(B,),
        in_specs=[
            pl.BlockSpec((1, D, T), lambda b: (b, 0, 0)),
            pl.BlockSpec((K, D), lambda b: (0, 0)),
        ],
        out_specs=[
            pl.BlockSpec((1, D, T), lambda b: (b, 0, 0)),
            pl.BlockSpec((1, 1, T), lambda b: (b, 0, 0)),
        ],
        out_shape=[
            jax.ShapeDtypeStruct((B, D, T), jnp.float32),
            jax.ShapeDtypeStruct((B, 1, T), jnp.int32),
        ],
    )(inputs, W)
    return (q, idx.reshape(B * T, 1))


# speedup vs baseline: 2.2187x; 1.3407x over previous
import jax
import jax.numpy as jnp
from jax.experimental import pallas as pl
from jax.experimental.pallas import tpu as pltpu


def _vq_body(x_ref, w_ref, q_ref, idx_ref):
    x = x_ref[0]
    w = w_ref[...]
    K = w.shape[0]
    TT = x.shape[1]

    s2 = jax.lax.dot_general(
        -2.0 * w, x, (((1,), (0,)), ((), ())),
        preferred_element_type=jnp.float32)
    wn = jnp.sum(w * w, axis=1, keepdims=True)
    xn = jnp.sum(x * x, axis=0, keepdims=True)
    dist = (wn + xn) + s2

    iota = jax.lax.broadcasted_iota(jnp.int32, (K, TT), 0)
    idx = jnp.argmin(dist, axis=0)[None, :].astype(jnp.int32)

    oh = (iota == idx).astype(jnp.float32)
    q = jax.lax.dot_general(
        w, oh, (((0,), (0,)), ((), ())),
        preferred_element_type=jnp.float32)

    q_ref[0] = x + (q - x)
    idx_ref[0] = idx


def kernel(inputs, W):
    B, D, T = inputs.shape
    K = W.shape[0]
    q, idx = pl.pallas_call(
        _vq_body,
        grid=(B,),
        in_specs=[
            pl.BlockSpec((1, D, T), lambda b: (b, 0, 0)),
            pl.BlockSpec((K, D), lambda b: (0, 0)),
        ],
        out_specs=[
            pl.BlockSpec((1, D, T), lambda b: (b, 0, 0)),
            pl.BlockSpec((1, 1, T), lambda b: (b, 0, 0)),
        ],
        out_shape=[
            jax.ShapeDtypeStruct((B, D, T), jnp.float32),
            jax.ShapeDtypeStruct((B, 1, T), jnp.int32),
        ],
        compiler_params=pltpu.CompilerParams(
            dimension_semantics=("parallel",)),
    )(inputs, W)
    return (q, idx.reshape(B * T, 1))
